# baseline (speedup 1.0000x reference)
import functools

import jax
import jax.numpy as jnp
from jax import lax
from jax.experimental import pallas as pl
from jax.experimental.pallas import tpu as pltpu
from jax.experimental.pallas import tpu_sc as plsc

NC = 2
NS = 16
NW = NC * NS
EMBED = 64

BATCH = 16384
HIST = 50
ROWS_PER_W = BATCH // NW
GB = 8
NGROUPS = ROWS_PER_W // GB


def _gather_body(idx_hbm, tab_hbm, out_hbm, idx_v, rows_a, rows_b, gsa, gsb, wsa, wsb):
    sid = lax.axis_index("s")
    cid = lax.axis_index("c")
    wid = sid * NC + cid
    b0 = wid * ROWS_PER_W
    pltpu.sync_copy(idx_hbm.at[pl.ds(b0, ROWS_PER_W)], idx_v)

    def fire_gather(g, buf, sem):
        for i in range(GB):
            pltpu.async_copy(tab_hbm.at[idx_v.at[g * GB + i]], buf.at[i], sem)

    def wait_gather(buf, sem):
        pltpu.make_async_copy(out_hbm.at[pl.ds(b0, GB)], buf, sem).wait()

    def fire_write(g, buf, sem):
        pltpu.async_copy(buf, out_hbm.at[pl.ds(b0 + g * GB, GB)], sem)

    def wait_write(buf, sem):
        pltpu.make_async_copy(buf, out_hbm.at[pl.ds(b0, GB)], sem).wait()

    fire_gather(0, rows_a, gsa)

    @pl.loop(0, NGROUPS, step=2)
    def _(g):
        pl.when(g > 0)(lambda: wait_write(rows_b, wsb))
        fire_gather(g + 1, rows_b, gsb)
        wait_gather(rows_a, gsa)
        fire_write(g, rows_a, wsa)
        wait_write(rows_a, wsa)
        pl.when(g + 2 < NGROUPS)(lambda: fire_gather(g + 2, rows_a, gsa))
        wait_gather(rows_b, gsb)
        fire_write(g + 1, rows_b, wsb)

    wait_write(rows_b, wsb)


_gather = functools.partial(
    pl.kernel,
    out_type=jax.ShapeDtypeStruct((BATCH, HIST, EMBED), jnp.float32),
    mesh=plsc.VectorSubcoreMesh(
        core_axis_name="c", subcore_axis_name="s", num_cores=NC, num_subcores=NS
    ),
    scratch_types=[
        pltpu.VMEM((ROWS_PER_W, HIST), jnp.int32),
        pltpu.VMEM((GB, HIST, EMBED), jnp.float32),
        pltpu.VMEM((GB, HIST, EMBED), jnp.float32),
        pltpu.SemaphoreType.DMA,
        pltpu.SemaphoreType.DMA,
        pltpu.SemaphoreType.DMA,
        pltpu.SemaphoreType.DMA,
    ],
    compiler_params=pltpu.CompilerParams(use_tc_tiling_on_sc=False),
)(_gather_body)


def kernel(input, weight):
    return _gather(input.astype(jnp.int32), weight)
